# Initial kernel scaffold; baseline (speedup 1.0000x reference)
import jax
import jax.numpy as jnp
from jax.experimental import pallas as pl


def kernel(senders, receivers, senders_batch, receivers_batch, params):
    raise NotImplementedError("write your pallas kernel here")


# speedup vs baseline: 1.0994x; 1.0994x over previous
import jax
import jax.numpy as jnp
from jax import lax
from jax.experimental import pallas as pl
from jax.experimental.pallas import tpu as pltpu
from jax.experimental.pallas import tpu_sc as plsc

B = 16384
N = 327680
D = 64
H = 64
TWO_N = 2 * N

NC = 2
NS = 16
NW = NC * NS

_SUB = 128
_CHA = 512
_KSUB = _CHA // _SUB


_PA = TWO_N // NW
_ITA = _PA // _CHA


def _gather_body(ids_hbm, emb_hbm, out_hbm, idx_v, rows_v, sem):
    c = lax.axis_index("c")
    s = lax.axis_index("s")
    wid = s * NC + c
    base = wid * _PA

    def it(i, carry):
        off = base + i * _CHA
        pltpu.sync_copy(ids_hbm.at[pl.ds(off, _CHA)], idx_v)
        descs = [
            pltpu.async_copy(
                emb_hbm.at[idx_v.at[pl.ds(k * _SUB, _SUB)]],
                rows_v.at[pl.ds(k * _SUB, _SUB)],
                sem)
            for k in range(_KSUB)
        ]
        for d_ in descs:
            d_.wait()
        pltpu.sync_copy(rows_v, out_hbm.at[pl.ds(off, _CHA)])
        return carry

    lax.fori_loop(0, _ITA, it, 0)


def _sc_gather(ids, emb):
    return pl.kernel(
        _gather_body,
        out_type=jax.ShapeDtypeStruct((TWO_N, D), jnp.float32),
        mesh=plsc.VectorSubcoreMesh(
            core_axis_name="c", subcore_axis_name="s",
            num_cores=NC, num_subcores=NS),
        compiler_params=pltpu.CompilerParams(use_tc_tiling_on_sc=False),
        scratch_types=[
            pltpu.VMEM((_CHA,), jnp.int32),
            pltpu.VMEM((_CHA, D), jnp.float32),
            pltpu.SemaphoreType.DMA,
        ],
    )(ids, emb)


_RB = 1024
_GB = N // _RB
_SIDE_BLOCKS = _GB // 2


def _phi_body(x_ref, w1_ref, b1_ref, w2_ref, b2_ref, o_ref):
    x = x_ref[...]
    h = jnp.maximum(
        jnp.dot(x, w1_ref[0], preferred_element_type=jnp.float32) + b1_ref[0],
        0.0)
    o_ref[...] = (
        jnp.dot(h, w2_ref[0], preferred_element_type=jnp.float32) + b2_ref[0])


def _tc_phi(x_p, w1p, b1p, w2p, b2p):
    side = lambda i: (i // _SIDE_BLOCKS, 0, 0)
    return pl.pallas_call(
        _phi_body,
        grid=(_GB,),
        in_specs=[
            pl.BlockSpec((_RB, 2 * D), lambda i: (i, 0)),
            pl.BlockSpec((1, 2 * D, 2 * H), side),
            pl.BlockSpec((1, 1, 2 * H), side),
            pl.BlockSpec((1, 2 * H, 2 * H), side),
            pl.BlockSpec((1, 1, 2 * H), side),
        ],
        out_specs=pl.BlockSpec((_RB, 2 * H), lambda i: (i, 0)),
        out_shape=jax.ShapeDtypeStruct((N, 2 * H), jnp.float32),
    )(x_p, w1p, b1p, w2p, b2p)


_PC = N // NS
_ITC = _PC // _CHA
_BZ = B // NS


def _pool_body(h2_hbm, seg_hbm, out_hbm, idx_v, rows_v, acc):
    s = lax.axis_index("s")

    def zr(i, carry):
        for j in range(D // 16):
            rows_v[i, pl.ds(16 * j, 16)] = jnp.zeros((16,), jnp.float32)
        return carry

    lax.fori_loop(0, _CHA, zr, 0)
    for t in range(_BZ // _CHA):
        pltpu.sync_copy(rows_v, acc.at[pl.ds(s * _BZ + t * _CHA, _CHA)])
    plsc.subcore_barrier()

    base = s * _PC

    def it(i, carry):
        off = base + i * _CHA
        pltpu.sync_copy(h2_hbm.at[pl.ds(off, _CHA)], rows_v)
        for k in range(_KSUB):
            pltpu.sync_copy(seg_hbm.at[pl.ds(off + k * _SUB, _SUB)],
                            idx_v.at[k])
            pltpu.sync_copy(rows_v.at[pl.ds(k * _SUB, _SUB)],
                            acc.at[idx_v.at[k]], add=True)
        return carry

    lax.fori_loop(0, _ITC, it, 0)
    plsc.subcore_barrier()

    for t in range(_BZ // _CHA):
        r0 = s * _BZ + t * _CHA
        pltpu.sync_copy(acc.at[pl.ds(r0, _CHA)], rows_v)
        pltpu.sync_copy(rows_v, out_hbm.at[pl.ds(r0, _CHA)])


def _sc_pool_side(h2_side, seg_side):
    return pl.kernel(
        _pool_body,
        out_type=jax.ShapeDtypeStruct((B, H), jnp.float32),
        mesh=plsc.VectorSubcoreMesh(
            core_axis_name="c", subcore_axis_name="s",
            num_cores=1, num_subcores=NS),
        compiler_params=pltpu.CompilerParams(use_tc_tiling_on_sc=False),
        scratch_types=[
            pltpu.VMEM((_KSUB, _SUB), jnp.int32),
            pltpu.VMEM((_CHA, H), jnp.float32),
            pltpu.VMEM_SHARED((B, H), jnp.float32),
        ],
    )(h2_side, seg_side)


_HB = 512
_HGB = (B // 2) // _HB


def _head_body(s_ref, r_ref, sw1, sb1, sw2, sb2, rw1, rb1, rw2, rb2,
               pa, pb, pb1, pw2, o_ref):
    def rho(x, w1, b1, w2, b2):
        t = jnp.maximum(
            jnp.dot(x, w1[...], preferred_element_type=jnp.float32) + b1[...],
            0.0)
        return jnp.dot(t, w2[...], preferred_element_type=jnp.float32) + b2[...]

    sr = rho(s_ref[...], sw1, sb1, sw2, sb2)
    rr = rho(r_ref[...], rw1, rb1, rw2, rb2)
    hh = jnp.maximum(
        jnp.dot(sr, pa[...], preferred_element_type=jnp.float32)
        + jnp.dot(rr, pb[...], preferred_element_type=jnp.float32)
        + pb1[...], 0.0)
    o_ref[...] = jnp.dot(hh, pw2[...], preferred_element_type=jnp.float32)


def _tc_head(s_p, r_p, args):
    full = lambda shape: pl.BlockSpec(shape, lambda i: tuple(0 for _ in shape))
    w_specs = [
        full((2 * H, 2 * H)), full((1, 2 * H)),
        full((2 * H, 2 * H)), full((1, 2 * H)),
        full((2 * H, 2 * H)), full((1, 2 * H)),
        full((2 * H, 2 * H)), full((1, 2 * H)),
        full((2 * H, 2 * H)), full((2 * H, 2 * H)), full((1, 2 * H)),
        full((2 * H, 8)),
    ]
    return pl.pallas_call(
        _head_body,
        grid=(_HGB,),
        in_specs=[pl.BlockSpec((_HB, 2 * H), lambda i: (i, 0)),
                  pl.BlockSpec((_HB, 2 * H), lambda i: (i, 0))] + w_specs,
        out_specs=pl.BlockSpec((_HB, 8), lambda i: (i, 0)),
        out_shape=jax.ShapeDtypeStruct((B // 2, 8), jnp.float32),
    )(s_p, r_p, *args)


def _blockdiag2(w):
    return jnp.kron(jnp.eye(2, dtype=w.dtype), w)


def _tile2(b):
    return jnp.tile(b.reshape(1, -1), (1, 2))


def kernel(senders, receivers, senders_batch, receivers_batch, params):
    p = params
    sp, rp = p['sender'], p['receiver']
    ids = jnp.concatenate([senders, receivers])
    seg = jnp.concatenate([senders_batch, receivers_batch])

    x = _sc_gather(ids, p['emb'])
    x_p = x.reshape(N, 2 * D)

    w1p = jnp.stack([_blockdiag2(sp['phi_W1']), _blockdiag2(rp['phi_W1'])])
    b1p = jnp.stack([_tile2(sp['phi_b1']), _tile2(rp['phi_b1'])])
    w2p = jnp.stack([_blockdiag2(sp['phi_W2']), _blockdiag2(rp['phi_W2'])])
    b2p = jnp.stack([_tile2(sp['phi_b2']), _tile2(rp['phi_b2'])])
    h2_p = _tc_phi(x_p, w1p, b1p, w2p, b2p)
    h2 = h2_p.reshape(TWO_N, H)

    pooled_s = _sc_pool_side(h2[:N], seg[:N])
    pooled_r = _sc_pool_side(h2[N:], seg[N:])
    s_pk = pooled_s.reshape(B // 2, 2 * H)
    r_pk = pooled_r.reshape(B // 2, 2 * H)

    pw2 = jnp.zeros((2 * H, 8), jnp.float32)
    pw2 = pw2.at[:H, 0].set(p['pred_W2'][:, 0])
    pw2 = pw2.at[H:, 1].set(p['pred_W2'][:, 0])

    head_args = (
        _blockdiag2(sp['rho_W1']), _tile2(sp['rho_b1']),
        _blockdiag2(sp['rho_W2']), _tile2(sp['rho_b2']),
        _blockdiag2(rp['rho_W1']), _tile2(rp['rho_b1']),
        _blockdiag2(rp['rho_W2']), _tile2(rp['rho_b2']),
        _blockdiag2(p['pred_W1'][:H]), _blockdiag2(p['pred_W1'][H:]),
        _tile2(p['pred_b1']),
        pw2,
    )
    out_pk = _tc_head(s_pk, r_pk, head_args)
    out = out_pk[:, :2].reshape(B, 1) + p['pred_b2'].reshape(1, 1)
    return out


# speedup vs baseline: 1.9618x; 1.7843x over previous
import jax
import jax.numpy as jnp
from jax import lax
from jax.experimental import pallas as pl
from jax.experimental.pallas import tpu as pltpu
from jax.experimental.pallas import tpu_sc as plsc

B = 16384
N = 327680
D = 64
H = 64

NC = 2
NS = 16
NW = NC * NS

_SUB = 128
_CHA = 512
_KSUB = _CHA // _SUB


_PA = N // NW
_ITA = _PA // _CHA


def _gather_body(ids_hbm, emb_hbm, out_hbm, idx_v, rows_v, sem_i, sem_g,
                 sem_w):
    c = lax.axis_index("c")
    s = lax.axis_index("s")
    base = (s * NC + c) * _PA

    def idx_copy(i):
        r = lax.rem(i, 2)
        return (ids_hbm.at[pl.ds(base + i * _CHA, _CHA)], idx_v.at[r], sem_i)

    def wb_copy(i):
        r = lax.rem(i, 2)
        return (rows_v.at[r], out_hbm.at[pl.ds(base + i * _CHA, _CHA)], sem_w)

    pltpu.async_copy(*idx_copy(0))

    def it(i, carry):
        r = lax.rem(i, 2)
        pltpu.make_async_copy(*idx_copy(i)).wait()

        @pl.when(i + 1 < _ITA)
        def _():
            pltpu.async_copy(*idx_copy(i + 1))

        @pl.when(i >= 2)
        def _():
            pltpu.make_async_copy(*wb_copy(i - 2)).wait()

        descs = [
            pltpu.async_copy(
                emb_hbm.at[idx_v.at[r, pl.ds(k * _SUB, _SUB)]],
                rows_v.at[r, pl.ds(k * _SUB, _SUB)],
                sem_g)
            for k in range(_KSUB)
        ]
        for d_ in descs:
            d_.wait()
        pltpu.async_copy(*wb_copy(i))
        return carry

    lax.fori_loop(0, _ITA, it, 0)
    pltpu.make_async_copy(*wb_copy(_ITA - 2)).wait()
    pltpu.make_async_copy(*wb_copy(_ITA - 1)).wait()


def _sc_gather(ids, emb):
    return pl.kernel(
        _gather_body,
        out_type=jax.ShapeDtypeStruct((N, D), jnp.float32),
        mesh=plsc.VectorSubcoreMesh(
            core_axis_name="c", subcore_axis_name="s",
            num_cores=NC, num_subcores=NS),
        compiler_params=pltpu.CompilerParams(use_tc_tiling_on_sc=False),
        scratch_types=[
            pltpu.VMEM((2, _CHA), jnp.int32),
            pltpu.VMEM((2, _CHA, D), jnp.float32),
            pltpu.SemaphoreType.DMA,
            pltpu.SemaphoreType.DMA,
            pltpu.SemaphoreType.DMA,
        ],
    )(ids, emb)


_RB = 1024
_GBS = (N // 2) // _RB


def _phi_body(x_ref, w1_ref, b1_ref, w2_ref, b2_ref, o_ref):
    x = x_ref[...]
    h = jnp.maximum(
        jnp.dot(x, w1_ref[...], preferred_element_type=jnp.float32)
        + b1_ref[...], 0.0)
    o_ref[...] = (
        jnp.dot(h, w2_ref[...], preferred_element_type=jnp.float32)
        + b2_ref[...])


def _tc_phi(x_p, w1p, b1p, w2p, b2p):
    full = lambda shape: pl.BlockSpec(shape, lambda i: tuple(0 for _ in shape))
    return pl.pallas_call(
        _phi_body,
        grid=(_GBS,),
        in_specs=[
            pl.BlockSpec((_RB, 2 * D), lambda i: (i, 0)),
            full((2 * D, 2 * H)),
            full((1, 2 * H)),
            full((2 * H, 2 * H)),
            full((1, 2 * H)),
        ],
        out_specs=pl.BlockSpec((_RB, 2 * H), lambda i: (i, 0)),
        out_shape=jax.ShapeDtypeStruct((N // 2, 2 * H), jnp.float32),
    )(x_p, w1p, b1p, w2p, b2p)


_CHP = 256
_KSP = _CHP // _SUB
_PC = N // NS
_ITC = _PC // _CHP
_BZ = B // NS


def _pool_body(h2_hbm, seg_hbm, out_hbm, idx_v, rows_v, acc, sem_p):
    s = lax.axis_index("s")
    base = s * _PC

    def rows_copy(i):
        r = lax.rem(i, 2)
        return (h2_hbm.at[pl.ds(base + i * _CHP, _CHP)], rows_v.at[r], sem_p)

    def idx_copy(i, k):
        r = lax.rem(i, 2)
        return (seg_hbm.at[pl.ds(base + i * _CHP + k * _SUB, _SUB)],
                idx_v.at[r, k], sem_p)

    def sc_copy(i, k):
        r = lax.rem(i, 2)
        return (rows_v.at[r, pl.ds(k * _SUB, _SUB)],
                acc.at[idx_v.at[r, k]])

    def start_pre(i):
        pltpu.async_copy(*rows_copy(i))
        for k in range(_KSP):
            pltpu.async_copy(*idx_copy(i, k))

    def wait_pre(i):
        pltpu.make_async_copy(*rows_copy(i)).wait()
        for k in range(_KSP):
            pltpu.make_async_copy(*idx_copy(i, k)).wait()

    def zr(i, carry):
        for j in range(D // 16):
            rows_v[0, i, pl.ds(16 * j, 16)] = jnp.zeros((16,), jnp.float32)
        return carry

    lax.fori_loop(0, _CHP, zr, 0)
    for t in range(_BZ // _CHP):
        pltpu.sync_copy(rows_v.at[0], acc.at[pl.ds(s * _BZ + t * _CHP, _CHP)])
    plsc.subcore_barrier()

    start_pre(0)

    def it(i, carry):
        wait_pre(i)

        @pl.when(i + 1 < _ITC)
        def _():
            start_pre(i + 1)

        for k in range(_KSP):
            pltpu.sync_copy(*sc_copy(i, k), add=True)
        return carry

    lax.fori_loop(0, _ITC, it, 0)
    plsc.subcore_barrier()

    for t in range(_BZ // _CHP):
        r0 = s * _BZ + t * _CHP
        pltpu.sync_copy(acc.at[pl.ds(r0, _CHP)], rows_v.at[0])
        pltpu.sync_copy(rows_v.at[0], out_hbm.at[pl.ds(r0, _CHP)])


def _sc_pool_side(h2_side, seg_side):
    return pl.kernel(
        _pool_body,
        out_type=jax.ShapeDtypeStruct((B, H), jnp.float32),
        mesh=plsc.VectorSubcoreMesh(
            core_axis_name="c", subcore_axis_name="s",
            num_cores=1, num_subcores=NS),
        compiler_params=pltpu.CompilerParams(use_tc_tiling_on_sc=False),
        scratch_types=[
            pltpu.VMEM((2, _KSP, _SUB), jnp.int32),
            pltpu.VMEM((2, _CHP, H), jnp.float32),
            pltpu.VMEM_SHARED((B, H), jnp.float32),
            pltpu.SemaphoreType.DMA,
        ],
    )(h2_side, seg_side)


_HB = 512
_HGB = (B // 2) // _HB


def _head_body(s_ref, r_ref, sw1, sb1, sw2, sb2, rw1, rb1, rw2, rb2,
               pa, pb, pb1, pw2, o_ref):
    def rho(x, w1, b1, w2, b2):
        t = jnp.maximum(
            jnp.dot(x, w1[...], preferred_element_type=jnp.float32) + b1[...],
            0.0)
        return jnp.dot(t, w2[...], preferred_element_type=jnp.float32) + b2[...]

    sr = rho(s_ref[...], sw1, sb1, sw2, sb2)
    rr = rho(r_ref[...], rw1, rb1, rw2, rb2)
    hh = jnp.maximum(
        jnp.dot(sr, pa[...], preferred_element_type=jnp.float32)
        + jnp.dot(rr, pb[...], preferred_element_type=jnp.float32)
        + pb1[...], 0.0)
    o_ref[...] = jnp.dot(hh, pw2[...], preferred_element_type=jnp.float32)


def _tc_head(s_p, r_p, args):
    full = lambda shape: pl.BlockSpec(shape, lambda i: tuple(0 for _ in shape))
    w_specs = [
        full((2 * H, 2 * H)), full((1, 2 * H)),
        full((2 * H, 2 * H)), full((1, 2 * H)),
        full((2 * H, 2 * H)), full((1, 2 * H)),
        full((2 * H, 2 * H)), full((1, 2 * H)),
        full((2 * H, 2 * H)), full((2 * H, 2 * H)), full((1, 2 * H)),
        full((2 * H, 8)),
    ]
    return pl.pallas_call(
        _head_body,
        grid=(_HGB,),
        in_specs=[pl.BlockSpec((_HB, 2 * H), lambda i: (i, 0)),
                  pl.BlockSpec((_HB, 2 * H), lambda i: (i, 0))] + w_specs,
        out_specs=pl.BlockSpec((_HB, 8), lambda i: (i, 0)),
        out_shape=jax.ShapeDtypeStruct((B // 2, 8), jnp.float32),
    )(s_p, r_p, *args)


def _blockdiag2(w):
    return jnp.kron(jnp.eye(2, dtype=w.dtype), w)


def _tile2(b):
    return jnp.tile(b.reshape(1, -1), (1, 2))


def kernel(senders, receivers, senders_batch, receivers_batch, params):
    p = params
    sp, rp = p['sender'], p['receiver']

    xs = _sc_gather(senders, p['emb'])
    xr = _sc_gather(receivers, p['emb'])

    h2s_p = _tc_phi(
        xs.reshape(N // 2, 2 * D),
        _blockdiag2(sp['phi_W1']), _tile2(sp['phi_b1']),
        _blockdiag2(sp['phi_W2']), _tile2(sp['phi_b2']))
    h2r_p = _tc_phi(
        xr.reshape(N // 2, 2 * D),
        _blockdiag2(rp['phi_W1']), _tile2(rp['phi_b1']),
        _blockdiag2(rp['phi_W2']), _tile2(rp['phi_b2']))

    pooled_s = _sc_pool_side(h2s_p.reshape(N, H), senders_batch)
    pooled_r = _sc_pool_side(h2r_p.reshape(N, H), receivers_batch)
    s_pk = pooled_s.reshape(B // 2, 2 * H)
    r_pk = pooled_r.reshape(B // 2, 2 * H)

    pw2 = jnp.zeros((2 * H, 8), jnp.float32)
    pw2 = pw2.at[:H, 0].set(p['pred_W2'][:, 0])
    pw2 = pw2.at[H:, 1].set(p['pred_W2'][:, 0])

    head_args = (
        _blockdiag2(sp['rho_W1']), _tile2(sp['rho_b1']),
        _blockdiag2(sp['rho_W2']), _tile2(sp['rho_b2']),
        _blockdiag2(rp['rho_W1']), _tile2(rp['rho_b1']),
        _blockdiag2(rp['rho_W2']), _tile2(rp['rho_b2']),
        _blockdiag2(p['pred_W1'][:H]), _blockdiag2(p['pred_W1'][H:]),
        _tile2(p['pred_b1']),
        pw2,
    )
    out_pk = _tc_head(s_pk, r_pk, head_args)
    out = out_pk[:, :2].reshape(B, 1) + p['pred_b2'].reshape(1, 1)
    return out
